# Initial kernel scaffold; baseline (speedup 1.0000x reference)
import jax
import jax.numpy as jnp
from jax.experimental import pallas as pl


def kernel(ids, table):
    raise NotImplementedError("write your pallas kernel here")


# speedup vs baseline: 9.1959x; 9.1959x over previous
import functools

import jax
import jax.numpy as jnp
from jax import lax
from jax.experimental import pallas as pl
from jax.experimental.pallas import tpu as pltpu
from jax.experimental.pallas import tpu_sc as plsc

_CHUNK = 128
_NBUF = 2


@functools.lru_cache(maxsize=None)
def _make_gather(num_workers, rows, table_rows, d):
    mesh = plsc.VectorSubcoreMesh(core_axis_name="c", subcore_axis_name="s")
    nc = mesh.num_cores
    out_rows = num_workers * rows * _CHUNK

    @functools.partial(
        pl.kernel,
        mesh=mesh,
        out_type=jax.ShapeDtypeStruct((out_rows, d), jnp.float32),
        compiler_params=pltpu.CompilerParams(use_tc_tiling_on_sc=False),
        scratch_types=[
            pltpu.VMEM((rows, _CHUNK), jnp.int32),
            pltpu.VMEM((_NBUF, _CHUNK, d), jnp.float32),
            pltpu.SemaphoreType.DMA,
            pltpu.SemaphoreType.DMA,
        ],
    )
    def gather(ids_hbm, table_hbm, out_hbm, idx_v, rows_v, sem0, sem1):
        wid = lax.axis_index("s") * nc + lax.axis_index("c")
        base = wid * rows * _CHUNK
        sems = (sem0, sem1)

        pltpu.sync_copy(ids_hbm.at[wid], idx_v)

        for b in range(_NBUF):
            pltpu.async_copy(table_hbm.at[idx_v.at[b]], rows_v.at[b], sems[b])

        def step(i, carry):
            g = i * _NBUF
            for b in range(_NBUF):
                j = g + b
                pltpu.make_async_copy(
                    table_hbm.at[idx_v.at[j]], rows_v.at[b], sems[b]
                ).wait()
                pltpu.sync_copy(
                    rows_v.at[b], out_hbm.at[pl.ds(base + j * _CHUNK, _CHUNK)]
                )

                @pl.when(j + _NBUF < rows)
                def _():
                    pltpu.async_copy(
                        table_hbm.at[idx_v.at[j + _NBUF]], rows_v.at[b], sems[b]
                    )

            return carry

        lax.fori_loop(0, rows // _NBUF, step, 0)

    return gather


def kernel(ids, table):
    input_shape = ids.shape
    d = table.shape[1]
    ids_flat = jnp.reshape(ids, (-1,)).astype(jnp.int32)
    n = ids_flat.shape[0]

    info = plsc.get_sparse_core_info()
    num_workers = info.num_cores * info.num_subcores
    assert n % (num_workers * _CHUNK) == 0
    rows = n // (num_workers * _CHUNK)
    assert rows % _NBUF == 0

    ids3 = jnp.reshape(ids_flat, (num_workers, rows, _CHUNK))
    out = _make_gather(num_workers, rows, table.shape[0], d)(ids3, table)
    return jnp.reshape(out, input_shape + (d,))


# speedup vs baseline: 9.3032x; 1.0117x over previous
import functools

import jax
import jax.numpy as jnp
from jax import lax
from jax.experimental import pallas as pl
from jax.experimental.pallas import tpu as pltpu
from jax.experimental.pallas import tpu_sc as plsc

_CHUNK = 128
_G = 5
_SUPER = _CHUNK * _G


@functools.lru_cache(maxsize=None)
def _make_gather(num_workers, rows, table_rows, d):
    mesh = plsc.VectorSubcoreMesh(core_axis_name="c", subcore_axis_name="s")
    nc = mesh.num_cores
    out_rows = num_workers * rows * _CHUNK
    nsuper = rows // _G

    @functools.partial(
        pl.kernel,
        mesh=mesh,
        out_type=jax.ShapeDtypeStruct((out_rows, d), jnp.float32),
        compiler_params=pltpu.CompilerParams(use_tc_tiling_on_sc=False),
        scratch_types=[
            pltpu.VMEM((rows, _CHUNK), jnp.int32),
            pltpu.VMEM((2, _SUPER, d), jnp.float32),
            pltpu.SemaphoreType.DMA,
            pltpu.SemaphoreType.DMA,
            pltpu.SemaphoreType.DMA,
            pltpu.SemaphoreType.DMA,
        ],
    )
    def gather(ids_hbm, table_hbm, out_hbm, idx_v, sbuf, sg0, sg1, sw0, sw1):
        wid = lax.axis_index("s") * nc + lax.axis_index("c")
        base = wid * rows * _CHUNK
        sgs = (sg0, sg1)
        sws = (sw0, sw1)

        pltpu.sync_copy(ids_hbm.at[wid], idx_v)

        def out_slice(t):
            return out_hbm.at[pl.ds(base + t * _SUPER, _SUPER)]

        def fire_gathers(t, buf):
            for g in range(_G):
                pltpu.async_copy(
                    table_hbm.at[idx_v.at[t * _G + g]],
                    sbuf.at[buf, pl.ds(g * _CHUNK, _CHUNK)],
                    sgs[buf],
                )

        def wait_gathers(t, buf):
            pltpu.make_async_copy(out_slice(t), sbuf.at[buf], sgs[buf]).wait()

        def wait_write(t, buf):
            pltpu.make_async_copy(out_slice(t), sbuf.at[buf], sws[buf]).wait()

        fire_gathers(0, 0)

        def step(i, carry):
            for s in range(2):
                t = i * 2 + s
                wait_gathers(t, s)
                pltpu.async_copy(sbuf.at[s], out_slice(t), sws[s])
                b = 1 - s
                if s == 0:
                    @pl.when(i >= 1)
                    def _():
                        wait_write(t - 1, b)

                    fire_gathers(t + 1, b)
                else:
                    wait_write(t - 1, b)

                    @pl.when(t + 1 < nsuper)
                    def _():
                        fire_gathers(t + 1, b)

            return carry

        lax.fori_loop(0, nsuper // 2, step, 0)
        wait_write(nsuper - 1, 1)

    return gather


def kernel(ids, table):
    input_shape = ids.shape
    d = table.shape[1]
    ids_flat = jnp.reshape(ids, (-1,)).astype(jnp.int32)
    n = ids_flat.shape[0]

    info = plsc.get_sparse_core_info()
    num_workers = info.num_cores * info.num_subcores
    assert n % (num_workers * _CHUNK) == 0
    rows = n // (num_workers * _CHUNK)
    assert rows % (2 * _G) == 0

    ids3 = jnp.reshape(ids_flat, (num_workers, rows, _CHUNK))
    out = _make_gather(num_workers, rows, table.shape[0], d)(ids3, table)
    return jnp.reshape(out, input_shape + (d,))
